# baseline (speedup 1.0000x reference)
import jax
import jax.numpy as jnp
from jax.experimental import pallas as pl
from jax.experimental.pallas import tpu as pltpu

_EPS = 1e-10
_MIN_EXP = -8
_NUM_EMB = 24
_OUT_D = 128
_EXP_D = 32
_LN10 = 2.302585092994046
_B = 64


def _body(x_ref, thl_ref, q2pad_ref, out_ref):
    x = x_ref[...]
    s = x.shape[1]
    e = jnp.floor(jnp.log10(x + _EPS))
    m2 = 2.0 * (x / jnp.exp(e * _LN10))
    idx = jnp.clip(e.astype(jnp.int32) - _MIN_EXP, 0, _NUM_EMB - 1)
    mt = jnp.transpose(m2)
    it = jnp.transpose(idx.astype(jnp.float32))
    q2 = q2pad_ref[...]
    thl = thl_ref[...]
    lanes = jax.lax.broadcasted_iota(jnp.int32, (s, 2 * _NUM_EMB), 1)
    for r in range(_B):
        mcol = jax.lax.broadcast_in_dim(mt[:, r], (s, _OUT_D), (0,))
        icol = it[:, r].astype(jnp.int32)
        icol = jax.lax.broadcast_in_dim(icol, (s, 2 * _NUM_EMB), (0,))
        onehot = ((lanes == icol) |
                  (lanes == icol + _NUM_EMB)).astype(jnp.bfloat16)
        exp_part = jax.lax.dot_general(
            onehot, thl, (((1,), (0,)), ((), ())),
            preferred_element_type=jnp.float32)
        t = mcol - q2
        out_ref[r] = exp_part + jnp.exp(-(t * t))


@jax.jit
def kernel(numbers, table, q_values):
    b, s = numbers.shape
    hi = table.astype(jnp.bfloat16)
    lo = (table - hi.astype(jnp.float32)).astype(jnp.bfloat16)
    thl = jnp.zeros((2 * _NUM_EMB, _OUT_D), jnp.bfloat16)
    thl = thl.at[:_NUM_EMB, :_EXP_D].set(hi).at[_NUM_EMB:, :_EXP_D].set(lo)
    q2pad = jnp.concatenate(
        [jnp.full((_EXP_D,), 1e30, jnp.float32), 2.0 * q_values]
    ).reshape(1, _OUT_D)

    grid = (b // _B,)
    out = pl.pallas_call(
        _body,
        grid=grid,
        in_specs=[
            pl.BlockSpec((_B, s), lambda i: (i, 0)),
            pl.BlockSpec((2 * _NUM_EMB, _OUT_D), lambda i: (0, 0)),
            pl.BlockSpec((1, _OUT_D), lambda i: (0, 0)),
        ],
        out_specs=pl.BlockSpec((_B, s, _OUT_D), lambda i: (i, 0, 0)),
        out_shape=jax.ShapeDtypeStruct((b, s, _OUT_D), jnp.float32),
        compiler_params=pltpu.CompilerParams(
            dimension_semantics=("arbitrary",)),
    )(numbers, thl, q2pad)
    return out


# speedup vs baseline: 1.0610x; 1.0610x over previous
import jax
import jax.numpy as jnp
from jax.experimental import pallas as pl
from jax.experimental.pallas import tpu as pltpu

_EPS = 1e-10
_MIN_EXP = -8
_NUM_EMB = 24
_OUT_D = 128
_EXP_D = 32
_LN10 = 2.302585092994046
_B = 128


def _body(x_ref, thl_ref, q2pad_ref, out_ref):
    x = x_ref[...]
    s = x.shape[1]
    e = jnp.floor(jnp.log10(x + _EPS))
    m2 = 2.0 * (x / jnp.exp(e * _LN10))
    idx = jnp.clip(e.astype(jnp.int32) - _MIN_EXP, 0, _NUM_EMB - 1)
    mt = jnp.transpose(m2)
    it = jnp.transpose(idx.astype(jnp.float32))
    q2 = q2pad_ref[...]
    thl = thl_ref[...]
    lanes = jax.lax.broadcasted_iota(jnp.int32, (s, 2 * _NUM_EMB), 1)
    for r in range(_B):
        mcol = jax.lax.broadcast_in_dim(mt[:, r], (s, _OUT_D), (0,))
        icol = it[:, r].astype(jnp.int32)
        icol = jax.lax.broadcast_in_dim(icol, (s, 2 * _NUM_EMB), (0,))
        onehot = ((lanes == icol) |
                  (lanes == icol + _NUM_EMB)).astype(jnp.bfloat16)
        exp_part = jax.lax.dot_general(
            onehot, thl, (((1,), (0,)), ((), ())),
            preferred_element_type=jnp.float32)
        t = mcol - q2
        out_ref[r] = exp_part + jnp.exp(-(t * t))


@jax.jit
def kernel(numbers, table, q_values):
    b, s = numbers.shape
    hi = table.astype(jnp.bfloat16)
    lo = (table - hi.astype(jnp.float32)).astype(jnp.bfloat16)
    thl = jnp.zeros((2 * _NUM_EMB, _OUT_D), jnp.bfloat16)
    thl = thl.at[:_NUM_EMB, :_EXP_D].set(hi).at[_NUM_EMB:, :_EXP_D].set(lo)
    q2pad = jnp.concatenate(
        [jnp.full((_EXP_D,), 1e30, jnp.float32), 2.0 * q_values]
    ).reshape(1, _OUT_D)

    grid = (b // _B,)
    out = pl.pallas_call(
        _body,
        grid=grid,
        in_specs=[
            pl.BlockSpec((_B, s), lambda i: (i, 0)),
            pl.BlockSpec((2 * _NUM_EMB, _OUT_D), lambda i: (0, 0)),
            pl.BlockSpec((1, _OUT_D), lambda i: (0, 0)),
        ],
        out_specs=pl.BlockSpec((_B, s, _OUT_D), lambda i: (i, 0, 0)),
        out_shape=jax.ShapeDtypeStruct((b, s, _OUT_D), jnp.float32),
        compiler_params=pltpu.CompilerParams(
            dimension_semantics=("arbitrary",)),
    )(numbers, thl, q2pad)
    return out


# speedup vs baseline: 1.0698x; 1.0082x over previous
import jax
import jax.numpy as jnp
from jax.experimental import pallas as pl
from jax.experimental.pallas import tpu as pltpu

_EPS = 1e-10
_MIN_EXP = -8
_NUM_EMB = 24
_OUT_D = 128
_EXP_D = 32
_LN10 = 2.302585092994046
_B = 256


def _body(x_ref, thl_ref, q2pad_ref, out_ref):
    x = x_ref[...]
    s = x.shape[1]
    e = jnp.floor(jnp.log10(x + _EPS))
    m2 = 2.0 * (x / jnp.exp(e * _LN10))
    idx = jnp.clip(e.astype(jnp.int32) - _MIN_EXP, 0, _NUM_EMB - 1)
    mt = jnp.transpose(m2)
    it = jnp.transpose(idx.astype(jnp.float32))
    q2 = q2pad_ref[...]
    thl = thl_ref[...]
    lanes = jax.lax.broadcasted_iota(jnp.int32, (s, 2 * _NUM_EMB), 1)
    for r in range(_B):
        mcol = jax.lax.broadcast_in_dim(mt[:, r], (s, _OUT_D), (0,))
        icol = it[:, r].astype(jnp.int32)
        icol = jax.lax.broadcast_in_dim(icol, (s, 2 * _NUM_EMB), (0,))
        onehot = ((lanes == icol) |
                  (lanes == icol + _NUM_EMB)).astype(jnp.bfloat16)
        exp_part = jax.lax.dot_general(
            onehot, thl, (((1,), (0,)), ((), ())),
            preferred_element_type=jnp.float32)
        t = mcol - q2
        out_ref[r] = exp_part + jnp.exp(-(t * t))


@jax.jit
def kernel(numbers, table, q_values):
    b, s = numbers.shape
    hi = table.astype(jnp.bfloat16)
    lo = (table - hi.astype(jnp.float32)).astype(jnp.bfloat16)
    thl = jnp.zeros((2 * _NUM_EMB, _OUT_D), jnp.bfloat16)
    thl = thl.at[:_NUM_EMB, :_EXP_D].set(hi).at[_NUM_EMB:, :_EXP_D].set(lo)
    q2pad = jnp.concatenate(
        [jnp.full((_EXP_D,), 1e30, jnp.float32), 2.0 * q_values]
    ).reshape(1, _OUT_D)

    grid = (b // _B,)
    out = pl.pallas_call(
        _body,
        grid=grid,
        in_specs=[
            pl.BlockSpec((_B, s), lambda i: (i, 0)),
            pl.BlockSpec((2 * _NUM_EMB, _OUT_D), lambda i: (0, 0)),
            pl.BlockSpec((1, _OUT_D), lambda i: (0, 0)),
        ],
        out_specs=pl.BlockSpec((_B, s, _OUT_D), lambda i: (i, 0, 0)),
        out_shape=jax.ShapeDtypeStruct((b, s, _OUT_D), jnp.float32),
        compiler_params=pltpu.CompilerParams(
            dimension_semantics=("arbitrary",)),
    )(numbers, thl, q2pad)
    return out
